# baseline (speedup 1.0000x reference)
import functools

import jax
import jax.numpy as jnp
from jax import lax
from jax.experimental import pallas as pl
from jax.experimental.pallas import tpu as pltpu
from jax.experimental.pallas import tpu_sc as plsc

N = 10000
NPAD = 10240
E = 320000
DIN = 128
DOUT = 128
NC = 2
NS = 16
NW = NC * NS
EPW = 10240
K = 128
CH = EPW // K
EPAD = NW * EPW
RPT = NPAD // NS
FK = 2
BLK = 1280


@functools.partial(
    pl.kernel,
    out_type=jax.ShapeDtypeStruct((NC, NPAD), jnp.float32),
    mesh=plsc.VectorSubcoreMesh(core_axis_name="c", subcore_axis_name="s"),
    scratch_types=[
        pltpu.VMEM((CH, K), jnp.int32),
        pltpu.VMEM((K,), jnp.float32),
        pltpu.VMEM_SHARED((NPAD,), jnp.float32),
    ],
)
def _deg_kernel(dst_hbm, z1_hbm, out_hbm, didx, ones_v, deg_sh):
    cid = lax.axis_index("c")
    sid = lax.axis_index("s")
    wid = cid * NS + sid
    pltpu.sync_copy(dst_hbm.at[wid], didx)
    pltpu.sync_copy(z1_hbm.at[pl.ds(sid * RPT, RPT)],
                    deg_sh.at[pl.ds(sid * RPT, RPT)])
    for j in range(K // 16):
        ones_v[pl.ds(16 * j, 16)] = jnp.ones((16,), jnp.float32)
    plsc.subcore_barrier()

    def body(c, carry):
        pltpu.sync_copy(ones_v, deg_sh.at[didx.at[c]], add=True)
        return carry

    lax.fori_loop(0, CH, body, 0)
    plsc.subcore_barrier()
    pltpu.sync_copy(deg_sh.at[pl.ds(sid * RPT, RPT)],
                    out_hbm.at[cid, pl.ds(sid * RPT, RPT)])


G = 40
NGRP = CH // G


@functools.partial(
    pl.kernel,
    out_type=jax.ShapeDtypeStruct((NC, NPAD, DIN), jnp.float32),
    mesh=plsc.VectorSubcoreMesh(core_axis_name="c", subcore_axis_name="s"),
    scratch_types=[
        pltpu.VMEM((G, K), jnp.int32),
        pltpu.VMEM((G, K), jnp.int32),
        pltpu.VMEM((FK * K, DIN), jnp.float32),
        pltpu.VMEM_SHARED((NPAD, DIN), jnp.float32),
        pltpu.SemaphoreType.DMA,
        pltpu.SemaphoreType.DMA,
        pltpu.SemaphoreType.DMA,
        pltpu.SemaphoreType.DMA,
    ],
)
def _agg_kernel(y_hbm, src_hbm, dst_hbm, z2_hbm, out_hbm,
                sidx, didx, rows, acc_sh, gsem0, gsem1, ssem0, ssem1):
    cid = lax.axis_index("c")
    sid = lax.axis_index("s")
    wid = cid * NS + sid
    gsems = (gsem0, gsem1)
    ssems = (ssem0, ssem1)
    pltpu.sync_copy(z2_hbm, acc_sh.at[pl.ds(sid * RPT, RPT)])
    plsc.subcore_barrier()

    def group(g, carry):
        pltpu.sync_copy(src_hbm.at[wid, pl.ds(g * G, G)], sidx)
        pltpu.sync_copy(dst_hbm.at[wid, pl.ds(g * G, G)], didx)
        for b in range(2):
            pltpu.async_copy(y_hbm.at[sidx.at[b]],
                             rows.at[pl.ds(b * K, K)], gsems[b])

        def body(k, carry2):
            for b in range(2):
                l = 2 * k + b
                rb = rows.at[pl.ds(b * K, K)]
                pltpu.make_async_copy(y_hbm.at[sidx.at[l]], rb,
                                      gsems[b]).wait()
                pltpu.async_copy(rb, acc_sh.at[didx.at[l]], ssems[b],
                                 add=True)
            for b in range(2):
                l = 2 * k + b
                rb = rows.at[pl.ds(b * K, K)]

                @pl.when(l + 2 < G)
                def _():
                    pltpu.make_async_copy(rb, acc_sh.at[didx.at[l]],
                                          ssems[b]).wait()
                    pltpu.async_copy(y_hbm.at[sidx.at[l + 2]], rb, gsems[b])

            return carry2

        lax.fori_loop(0, G // 2, body, carry)
        for b in range(2):
            l = G - 2 + b
            pltpu.make_async_copy(rows.at[pl.ds(b * K, K)],
                                  acc_sh.at[didx.at[l]], ssems[b]).wait()
        return carry

    lax.fori_loop(0, NGRP, group, 0)
    plsc.subcore_barrier()
    pltpu.sync_copy(acc_sh.at[pl.ds(sid * RPT, RPT)],
                    out_hbm.at[cid, pl.ds(sid * RPT, RPT)])


def _scale_body(x_ref, dp_ref, y_ref):
    dp = dp_ref[...]
    deg = dp[0] + dp[1]
    dis = jnp.where(deg > 0, lax.rsqrt(deg), 0.0)
    y_ref[...] = x_ref[...] * dis


_scale = pl.pallas_call(
    _scale_body,
    grid=(NPAD // BLK,),
    in_specs=[
        pl.BlockSpec((BLK, DIN), lambda i: (i, 0)),
        pl.BlockSpec((NC, BLK, 1), lambda i: (0, i, 0)),
    ],
    out_specs=pl.BlockSpec((BLK, DIN), lambda i: (i, 0)),
    out_shape=jax.ShapeDtypeStruct((NPAD, DIN), jnp.float32),
)


def _final_body(ap_ref, dp_ref, w_ref, b_ref, o_ref):
    ap = ap_ref[...]
    acc = ap[0] + ap[1]
    dp = dp_ref[...]
    deg = dp[0] + dp[1]
    dis = jnp.where(deg > 0, lax.rsqrt(deg), 0.0)
    t = acc * dis
    o_ref[...] = (
        jnp.dot(t, w_ref[...], preferred_element_type=jnp.float32) + b_ref[...]
    )


_final = pl.pallas_call(
    _final_body,
    grid=(NPAD // BLK,),
    in_specs=[
        pl.BlockSpec((NC, BLK, DIN), lambda i: (0, i, 0)),
        pl.BlockSpec((NC, BLK, 1), lambda i: (0, i, 0)),
        pl.BlockSpec((DIN, DOUT), lambda i: (0, 0)),
        pl.BlockSpec((1, DOUT), lambda i: (0, 0)),
    ],
    out_specs=pl.BlockSpec((BLK, DOUT), lambda i: (i, 0)),
    out_shape=jax.ShapeDtypeStruct((NPAD, DOUT), jnp.float32),
)


def kernel(x, edge_index, W, b):
    x = x.astype(jnp.float32)
    src = edge_index[0].astype(jnp.int32)
    dst = edge_index[1].astype(jnp.int32)
    npad = EPAD - E
    pidx = jnp.arange(npad, dtype=jnp.int32)
    srcp = jnp.concatenate([src, N + pidx % (NPAD - N)])
    dstp = jnp.concatenate([dst, N + pidx % (NPAD - N)])
    src_r = srcp.reshape(NW, CH, K)
    dst_r = dstp.reshape(NW, CH, K)
    z1 = jnp.zeros((NPAD,), jnp.float32)
    z2 = jnp.zeros((RPT, DIN), jnp.float32)
    xp = jnp.concatenate([x, jnp.zeros((NPAD - N, DIN), jnp.float32)])

    deg_p = _deg_kernel(dst_r, z1)
    dp3 = deg_p.reshape(NC, NPAD, 1)
    y = _scale(xp, dp3)
    acc_p = _agg_kernel(y, src_r, dst_r, z2)
    out = _final(acc_p, dp3, W, b.reshape(1, DOUT))
    return out[:N]


# speedup vs baseline: 1.2068x; 1.2068x over previous
import functools

import jax
import jax.numpy as jnp
from jax import lax
from jax.experimental import pallas as pl
from jax.experimental.pallas import tpu as pltpu
from jax.experimental.pallas import tpu_sc as plsc

N = 10000
NPAD = 10240
E = 320000
DIN = 128
DOUT = 128
NC = 2
NS = 16
NW = NC * NS
EPW = 10240
K = 128
CH = EPW // K
EPAD = NW * EPW
RPT = NPAD // NS
FK = 2
BLK = 1280


@functools.partial(
    pl.kernel,
    out_type=jax.ShapeDtypeStruct((NC, NPAD), jnp.float32),
    mesh=plsc.VectorSubcoreMesh(core_axis_name="c", subcore_axis_name="s"),
    scratch_types=[
        pltpu.VMEM((CH, K), jnp.int32),
        pltpu.VMEM((K,), jnp.float32),
        pltpu.VMEM_SHARED((NPAD,), jnp.float32),
    ],
)
def _deg_kernel(dst_hbm, z1_hbm, out_hbm, didx, ones_v, deg_sh):
    cid = lax.axis_index("c")
    sid = lax.axis_index("s")
    wid = cid * NS + sid
    pltpu.sync_copy(dst_hbm.at[wid], didx)
    pltpu.sync_copy(z1_hbm.at[pl.ds(sid * RPT, RPT)],
                    deg_sh.at[pl.ds(sid * RPT, RPT)])
    for j in range(K // 16):
        ones_v[pl.ds(16 * j, 16)] = jnp.ones((16,), jnp.float32)
    plsc.subcore_barrier()

    def body(c, carry):
        pltpu.sync_copy(ones_v, deg_sh.at[didx.at[c]], add=True)
        return carry

    lax.fori_loop(0, CH, body, 0)
    plsc.subcore_barrier()
    pltpu.sync_copy(deg_sh.at[pl.ds(sid * RPT, RPT)],
                    out_hbm.at[cid, pl.ds(sid * RPT, RPT)])


G = 40
NGRP = CH // G


@functools.partial(
    pl.kernel,
    out_type=jax.ShapeDtypeStruct((NC, NPAD, DIN), jnp.float32),
    mesh=plsc.VectorSubcoreMesh(core_axis_name="c", subcore_axis_name="s"),
    scratch_types=[
        pltpu.VMEM((G, K), jnp.int32),
        pltpu.VMEM((G, K), jnp.int32),
        pltpu.VMEM((FK * K, DIN), jnp.float32),
        pltpu.VMEM_SHARED((NPAD, DIN), jnp.float32),
        pltpu.SemaphoreType.DMA,
        pltpu.SemaphoreType.DMA,
        pltpu.SemaphoreType.DMA,
        pltpu.SemaphoreType.DMA,
    ],
)
def _agg_kernel(y_hbm, src_hbm, dst_hbm, z2_hbm, out_hbm,
                sidx, didx, rows, acc_sh, gsem0, gsem1, ssem0, ssem1):
    cid = lax.axis_index("c")
    sid = lax.axis_index("s")
    wid = cid * NS + sid
    gsems = (gsem0, gsem1)
    ssems = (ssem0, ssem1)
    pltpu.sync_copy(z2_hbm, acc_sh.at[pl.ds(sid * RPT, RPT)])
    plsc.subcore_barrier()

    def group(g, carry):
        pltpu.sync_copy(src_hbm.at[wid, pl.ds(g * G, G)], sidx)
        pltpu.sync_copy(dst_hbm.at[wid, pl.ds(g * G, G)], didx)
        for b in range(2):
            pltpu.async_copy(y_hbm.at[sidx.at[b]],
                             rows.at[pl.ds(b * K, K)], gsems[b])

        def body(k, carry2):
            for b in range(2):
                l = 2 * k + b
                rb = rows.at[pl.ds(b * K, K)]
                pltpu.make_async_copy(y_hbm.at[sidx.at[l]], rb,
                                      gsems[b]).wait()
                pltpu.async_copy(rb, acc_sh.at[didx.at[l]], ssems[b],
                                 add=True)

                @pl.when(l + 2 < G)
                def _():
                    pltpu.make_async_copy(rb, acc_sh.at[didx.at[l]],
                                          ssems[b]).wait()
                    pltpu.async_copy(y_hbm.at[sidx.at[l + 2]], rb, gsems[b])

            return carry2

        lax.fori_loop(0, G // 2, body, carry)
        for b in range(2):
            l = G - 2 + b
            pltpu.make_async_copy(rows.at[pl.ds(b * K, K)],
                                  acc_sh.at[didx.at[l]], ssems[b]).wait()
        return carry

    lax.fori_loop(0, NGRP, group, 0)
    plsc.subcore_barrier()
    pltpu.sync_copy(acc_sh.at[pl.ds(sid * RPT, RPT)],
                    out_hbm.at[cid, pl.ds(sid * RPT, RPT)])


def _scale_body(x_ref, dp_ref, y_ref):
    dp = dp_ref[...]
    deg = dp[0] + dp[1]
    dis = jnp.where(deg > 0, lax.rsqrt(deg), 0.0)
    y_ref[...] = x_ref[...] * dis


SBLK = 1000

_scale = pl.pallas_call(
    _scale_body,
    grid=(N // SBLK,),
    in_specs=[
        pl.BlockSpec((SBLK, DIN), lambda i: (i, 0)),
        pl.BlockSpec((NC, SBLK, 1), lambda i: (0, i, 0)),
    ],
    out_specs=pl.BlockSpec((SBLK, DIN), lambda i: (i, 0)),
    out_shape=jax.ShapeDtypeStruct((N, DIN), jnp.float32),
)


def _final_body(ap_ref, dp_ref, w_ref, b_ref, o_ref):
    ap = ap_ref[...]
    acc = ap[0] + ap[1]
    dp = dp_ref[...]
    deg = dp[0] + dp[1]
    dis = jnp.where(deg > 0, lax.rsqrt(deg), 0.0)
    t = acc * dis
    o_ref[...] = (
        jnp.dot(t, w_ref[...], preferred_element_type=jnp.float32) + b_ref[...]
    )


_final = pl.pallas_call(
    _final_body,
    grid=(N // SBLK,),
    in_specs=[
        pl.BlockSpec((NC, SBLK, DIN), lambda i: (0, i, 0)),
        pl.BlockSpec((NC, SBLK, 1), lambda i: (0, i, 0)),
        pl.BlockSpec((DIN, DOUT), lambda i: (0, 0)),
        pl.BlockSpec((1, DOUT), lambda i: (0, 0)),
    ],
    out_specs=pl.BlockSpec((SBLK, DOUT), lambda i: (i, 0)),
    out_shape=jax.ShapeDtypeStruct((N, DOUT), jnp.float32),
)


def kernel(x, edge_index, W, b):
    x = x.astype(jnp.float32)
    src = edge_index[0].astype(jnp.int32)
    dst = edge_index[1].astype(jnp.int32)
    npad = EPAD - E
    pidx = jnp.arange(npad, dtype=jnp.int32)
    srcp = jnp.concatenate([src, pidx % N])
    dstp = jnp.concatenate([dst, N + pidx % (NPAD - N)])
    src_r = srcp.reshape(NW, CH, K)
    dst_r = dstp.reshape(NW, CH, K)
    z1 = jnp.zeros((NPAD,), jnp.float32)
    z2 = jnp.zeros((RPT, DIN), jnp.float32)

    deg_p = _deg_kernel(dst_r, z1)
    dp3 = deg_p.reshape(NC, NPAD, 1)
    y = _scale(x, dp3)
    acc_p = _agg_kernel(y, src_r, dst_r, z2)
    return _final(acc_p, dp3, W, b.reshape(1, DOUT))


# speedup vs baseline: 1.2363x; 1.0244x over previous
import functools

import jax
import jax.numpy as jnp
from jax import lax
from jax.experimental import pallas as pl
from jax.experimental.pallas import tpu as pltpu
from jax.experimental.pallas import tpu_sc as plsc

N = 10000
NPAD = 10240
E = 320000
DIN = 128
DOUT = 128
NC = 2
NS = 16
NW = NC * NS
EPW = 10240
K = 128
CH = EPW // K
EPAD = NW * EPW
RPT = NPAD // NS
FK = 2
BLK = 1280


@functools.partial(
    pl.kernel,
    out_type=jax.ShapeDtypeStruct((NC, NPAD), jnp.float32),
    mesh=plsc.VectorSubcoreMesh(core_axis_name="c", subcore_axis_name="s"),
    scratch_types=[
        pltpu.VMEM((CH, K), jnp.int32),
        pltpu.VMEM((K,), jnp.float32),
        pltpu.VMEM_SHARED((NPAD,), jnp.float32),
        pltpu.SemaphoreType.DMA,
    ],
)
def _deg_kernel(dst_hbm, z1_hbm, out_hbm, didx, ones_v, deg_sh, dsem):
    cid = lax.axis_index("c")
    sid = lax.axis_index("s")
    wid = cid * NS + sid
    pltpu.sync_copy(dst_hbm.at[wid], didx)
    pltpu.sync_copy(z1_hbm.at[pl.ds(sid * RPT, RPT)],
                    deg_sh.at[pl.ds(sid * RPT, RPT)])
    for j in range(K // 16):
        ones_v[pl.ds(16 * j, 16)] = jnp.ones((16,), jnp.float32)
    plsc.subcore_barrier()

    def body(gi, carry):
        for j in range(8):
            c = gi * 8 + j
            pltpu.async_copy(ones_v, deg_sh.at[didx.at[c]], dsem, add=True)
        for j in range(8):
            c = gi * 8 + j
            pltpu.make_async_copy(ones_v, deg_sh.at[didx.at[c]],
                                  dsem).wait()
        return carry

    lax.fori_loop(0, CH // 8, body, 0)
    plsc.subcore_barrier()
    pltpu.sync_copy(deg_sh.at[pl.ds(sid * RPT, RPT)],
                    out_hbm.at[cid, pl.ds(sid * RPT, RPT)])


G = 40
NGRP = CH // G


@functools.partial(
    pl.kernel,
    out_type=jax.ShapeDtypeStruct((NC, NPAD, DIN), jnp.float32),
    mesh=plsc.VectorSubcoreMesh(core_axis_name="c", subcore_axis_name="s"),
    scratch_types=[
        pltpu.VMEM((G, K), jnp.int32),
        pltpu.VMEM((G, K), jnp.int32),
        pltpu.VMEM((FK * K, DIN), jnp.float32),
        pltpu.VMEM_SHARED((NPAD, DIN), jnp.float32),
        pltpu.SemaphoreType.DMA,
        pltpu.SemaphoreType.DMA,
        pltpu.SemaphoreType.DMA,
        pltpu.SemaphoreType.DMA,
    ],
)
def _agg_kernel(y_hbm, src_hbm, dst_hbm, z2_hbm, out_hbm,
                sidx, didx, rows, acc_sh, gsem0, gsem1, ssem0, ssem1):
    cid = lax.axis_index("c")
    sid = lax.axis_index("s")
    wid = cid * NS + sid
    gsems = (gsem0, gsem1)
    ssems = (ssem0, ssem1)
    pltpu.sync_copy(z2_hbm, acc_sh.at[pl.ds(sid * RPT, RPT)])
    plsc.subcore_barrier()

    def group(g, carry):
        pltpu.sync_copy(src_hbm.at[wid, pl.ds(g * G, G)], sidx)
        pltpu.sync_copy(dst_hbm.at[wid, pl.ds(g * G, G)], didx)
        for b in range(2):
            pltpu.async_copy(y_hbm.at[sidx.at[b]],
                             rows.at[pl.ds(b * K, K)], gsems[b])

        def body(k, carry2):
            for b in range(2):
                l = 2 * k + b
                rb = rows.at[pl.ds(b * K, K)]
                pltpu.make_async_copy(y_hbm.at[sidx.at[l]], rb,
                                      gsems[b]).wait()
                pltpu.async_copy(rb, acc_sh.at[didx.at[l]], ssems[b],
                                 add=True)

                @pl.when(l + 2 < G)
                def _():
                    pltpu.make_async_copy(rb, acc_sh.at[didx.at[l]],
                                          ssems[b]).wait()
                    pltpu.async_copy(y_hbm.at[sidx.at[l + 2]], rb, gsems[b])

            return carry2

        lax.fori_loop(0, G // 2, body, carry)
        for b in range(2):
            l = G - 2 + b
            pltpu.make_async_copy(rows.at[pl.ds(b * K, K)],
                                  acc_sh.at[didx.at[l]], ssems[b]).wait()
        return carry

    lax.fori_loop(0, NGRP, group, 0)
    plsc.subcore_barrier()
    pltpu.sync_copy(acc_sh.at[pl.ds(sid * RPT, RPT)],
                    out_hbm.at[cid, pl.ds(sid * RPT, RPT)])


def _scale_body(x_ref, dp_ref, y_ref):
    dp = dp_ref[...]
    deg = dp[0] + dp[1]
    dis = jnp.where(deg > 0, lax.rsqrt(deg), 0.0)
    y_ref[...] = x_ref[...] * dis


SBLK = 1000

_scale = pl.pallas_call(
    _scale_body,
    grid=(N // SBLK,),
    in_specs=[
        pl.BlockSpec((SBLK, DIN), lambda i: (i, 0)),
        pl.BlockSpec((NC, SBLK, 1), lambda i: (0, i, 0)),
    ],
    out_specs=pl.BlockSpec((SBLK, DIN), lambda i: (i, 0)),
    out_shape=jax.ShapeDtypeStruct((N, DIN), jnp.float32),
)


def _final_body(ap_ref, dp_ref, w_ref, b_ref, o_ref):
    ap = ap_ref[...]
    acc = ap[0] + ap[1]
    dp = dp_ref[...]
    deg = dp[0] + dp[1]
    dis = jnp.where(deg > 0, lax.rsqrt(deg), 0.0)
    t = acc * dis
    o_ref[...] = (
        jnp.dot(t, w_ref[...], preferred_element_type=jnp.float32) + b_ref[...]
    )


_final = pl.pallas_call(
    _final_body,
    grid=(N // SBLK,),
    in_specs=[
        pl.BlockSpec((NC, SBLK, DIN), lambda i: (0, i, 0)),
        pl.BlockSpec((NC, SBLK, 1), lambda i: (0, i, 0)),
        pl.BlockSpec((DIN, DOUT), lambda i: (0, 0)),
        pl.BlockSpec((1, DOUT), lambda i: (0, 0)),
    ],
    out_specs=pl.BlockSpec((SBLK, DOUT), lambda i: (i, 0)),
    out_shape=jax.ShapeDtypeStruct((N, DOUT), jnp.float32),
)


def kernel(x, edge_index, W, b):
    x = x.astype(jnp.float32)
    src = edge_index[0].astype(jnp.int32)
    dst = edge_index[1].astype(jnp.int32)
    npad = EPAD - E
    pidx = jnp.arange(npad, dtype=jnp.int32)
    srcp = jnp.concatenate([src, pidx % N])
    dstp = jnp.concatenate([dst, N + pidx % (NPAD - N)])
    src_r = srcp.reshape(NW, CH, K)
    dst_r = dstp.reshape(NW, CH, K)
    z1 = jnp.zeros((NPAD,), jnp.float32)
    z2 = jnp.zeros((RPT, DIN), jnp.float32)

    deg_p = _deg_kernel(dst_r, z1)
    dp3 = deg_p.reshape(NC, NPAD, 1)
    y = _scale(x, dp3)
    acc_p = _agg_kernel(y, src_r, dst_r, z2)
    return _final(acc_p, dp3, W, b.reshape(1, DOUT))


# speedup vs baseline: 1.2406x; 1.0035x over previous
import functools

import jax
import jax.numpy as jnp
from jax import lax
from jax.experimental import pallas as pl
from jax.experimental.pallas import tpu as pltpu
from jax.experimental.pallas import tpu_sc as plsc

N = 10000
NPAD = 10240
E = 320000
DIN = 128
DOUT = 128
NC = 2
NS = 16
NW = NC * NS
EPW = 10240
K = 128
CH = EPW // K
EPAD = NW * EPW
RPT = NPAD // NS
FK = 2


@functools.partial(
    pl.kernel,
    out_type=jax.ShapeDtypeStruct((NC, NPAD), jnp.float32),
    mesh=plsc.VectorSubcoreMesh(core_axis_name="c", subcore_axis_name="s"),
    scratch_types=[
        pltpu.VMEM((CH, K), jnp.int32),
        pltpu.VMEM((K,), jnp.float32),
        pltpu.VMEM_SHARED((NPAD,), jnp.float32),
        pltpu.SemaphoreType.DMA,
    ],
)
def _deg_kernel(dst_hbm, z1_hbm, out_hbm, didx, ones_v, deg_sh, dsem):
    cid = lax.axis_index("c")
    sid = lax.axis_index("s")
    wid = cid * NS + sid
    pltpu.sync_copy(dst_hbm.at[wid], didx)
    pltpu.sync_copy(z1_hbm.at[pl.ds(sid * RPT, RPT)],
                    deg_sh.at[pl.ds(sid * RPT, RPT)])
    for j in range(K // 16):
        ones_v[pl.ds(16 * j, 16)] = jnp.ones((16,), jnp.float32)
    plsc.subcore_barrier()

    def body(gi, carry):
        for j in range(8):
            c = gi * 8 + j
            pltpu.async_copy(ones_v, deg_sh.at[didx.at[c]], dsem, add=True)
        for j in range(8):
            c = gi * 8 + j
            pltpu.make_async_copy(ones_v, deg_sh.at[didx.at[c]],
                                  dsem).wait()
        return carry

    lax.fori_loop(0, CH // 8, body, 0)
    plsc.subcore_barrier()
    pltpu.sync_copy(deg_sh.at[pl.ds(sid * RPT, RPT)],
                    out_hbm.at[cid, pl.ds(sid * RPT, RPT)])


G = 40
NGRP = CH // G


@functools.partial(
    pl.kernel,
    out_type=jax.ShapeDtypeStruct((NC, NPAD, DIN), jnp.float32),
    mesh=plsc.VectorSubcoreMesh(core_axis_name="c", subcore_axis_name="s"),
    scratch_types=[
        pltpu.VMEM((G, K), jnp.int32),
        pltpu.VMEM((G, K), jnp.int32),
        pltpu.VMEM((FK * K, DIN), jnp.float32),
        pltpu.VMEM_SHARED((NPAD, DIN), jnp.float32),
        pltpu.SemaphoreType.DMA,
        pltpu.SemaphoreType.DMA,
        pltpu.SemaphoreType.DMA,
        pltpu.SemaphoreType.DMA,
    ],
)
def _agg_kernel(y_hbm, src_hbm, dst_hbm, z2_hbm, out_hbm,
                sidx, didx, rows, acc_sh, gsem0, gsem1, ssem0, ssem1):
    cid = lax.axis_index("c")
    sid = lax.axis_index("s")
    wid = cid * NS + sid
    gsems = (gsem0, gsem1)
    ssems = (ssem0, ssem1)
    pltpu.sync_copy(z2_hbm, acc_sh.at[pl.ds(sid * RPT, RPT)])
    plsc.subcore_barrier()

    def group(g, carry):
        pltpu.sync_copy(src_hbm.at[wid, pl.ds(g * G, G)], sidx)
        pltpu.sync_copy(dst_hbm.at[wid, pl.ds(g * G, G)], didx)
        for b in range(2):
            pltpu.async_copy(y_hbm.at[sidx.at[b]],
                             rows.at[pl.ds(b * K, K)], gsems[b])

        def body(k, carry2):
            for b in range(2):
                l = 2 * k + b
                rb = rows.at[pl.ds(b * K, K)]
                pltpu.make_async_copy(y_hbm.at[sidx.at[l]], rb,
                                      gsems[b]).wait()
                pltpu.async_copy(rb, acc_sh.at[didx.at[l]], ssems[b],
                                 add=True)

                @pl.when(l + 2 < G)
                def _():
                    pltpu.make_async_copy(rb, acc_sh.at[didx.at[l]],
                                          ssems[b]).wait()
                    pltpu.async_copy(y_hbm.at[sidx.at[l + 2]], rb, gsems[b])

            return carry2

        lax.fori_loop(0, G // 2, body, carry)
        for b in range(2):
            l = G - 2 + b
            pltpu.make_async_copy(rows.at[pl.ds(b * K, K)],
                                  acc_sh.at[didx.at[l]], ssems[b]).wait()
        return carry

    lax.fori_loop(0, NGRP, group, 0)
    plsc.subcore_barrier()
    pltpu.sync_copy(acc_sh.at[pl.ds(sid * RPT, RPT)],
                    out_hbm.at[cid, pl.ds(sid * RPT, RPT)])


def _scale_body(x_ref, dp_ref, y_ref):
    dp = dp_ref[...]
    deg = dp[0] + dp[1]
    dis = jnp.where(deg > 0, lax.rsqrt(deg), 0.0)
    y_ref[...] = x_ref[...] * dis


SBLK = 1000

_scale = pl.pallas_call(
    _scale_body,
    grid=(N // SBLK,),
    in_specs=[
        pl.BlockSpec((SBLK, DIN), lambda i: (i, 0)),
        pl.BlockSpec((NC, SBLK, 1), lambda i: (0, i, 0)),
    ],
    out_specs=pl.BlockSpec((SBLK, DIN), lambda i: (i, 0)),
    out_shape=jax.ShapeDtypeStruct((N, DIN), jnp.float32),
)


def _final_body(ap_ref, dp_ref, w_ref, b_ref, o_ref):
    ap = ap_ref[...]
    acc = ap[0] + ap[1]
    dp = dp_ref[...]
    deg = dp[0] + dp[1]
    dis = jnp.where(deg > 0, lax.rsqrt(deg), 0.0)
    t = acc * dis
    o_ref[...] = (
        jnp.dot(t, w_ref[...], preferred_element_type=jnp.float32) + b_ref[...]
    )


_final = pl.pallas_call(
    _final_body,
    grid=(N // SBLK,),
    in_specs=[
        pl.BlockSpec((NC, SBLK, DIN), lambda i: (0, i, 0)),
        pl.BlockSpec((NC, SBLK, 1), lambda i: (0, i, 0)),
        pl.BlockSpec((DIN, DOUT), lambda i: (0, 0)),
        pl.BlockSpec((1, DOUT), lambda i: (0, 0)),
    ],
    out_specs=pl.BlockSpec((SBLK, DOUT), lambda i: (i, 0)),
    out_shape=jax.ShapeDtypeStruct((N, DOUT), jnp.float32),
)


def kernel(x, edge_index, W, b):
    x = x.astype(jnp.float32)
    src = edge_index[0].astype(jnp.int32)
    dst = edge_index[1].astype(jnp.int32)
    npad = EPAD - E
    pidx = jnp.arange(npad, dtype=jnp.int32)
    srcp = jnp.concatenate([src, pidx % N])
    dstp = jnp.concatenate([dst, N + pidx % (NPAD - N)])
    src_r = srcp.reshape(NW, CH, K)
    dst_r = dstp.reshape(NW, CH, K)
    z1 = jnp.zeros((NPAD,), jnp.float32)
    z2 = jnp.zeros((RPT, DIN), jnp.float32)

    deg_p = _deg_kernel(dst_r, z1)
    dp3 = deg_p.reshape(NC, NPAD, 1)
    y = _scale(x, dp3)
    acc_p = _agg_kernel(y, src_r, dst_r, z2)
    return _final(acc_p, dp3, W, b.reshape(1, DOUT))
